# Initial kernel scaffold; baseline (speedup 1.0000x reference)
import jax
import jax.numpy as jnp
from jax.experimental import pallas as pl


def kernel(x, edge_index, batch, W1, b1, W2, b2, W3, b3, M1, mb1, M2, mb2, Wrel, brel, Wroot, P1, pb1, P2, pb2):
    raise NotImplementedError("write your pallas kernel here")


# speedup vs baseline: 1.1596x; 1.1596x over previous
import functools

import jax
import jax.numpy as jnp
from jax import lax
from jax.experimental import pallas as pl

N_NODES = 10000
NPAD = 10240
B = 64


def _mm_bias_kernel(a_ref, w_ref, b_ref, o_ref):
    o_ref[...] = jnp.dot(a_ref[...], w_ref[...],
                         preferred_element_type=jnp.float32) + b_ref[...]


def _relu_mm_kernel(s_ref, b_ref, w_ref, o_ref):
    h = jnp.maximum(s_ref[...] + b_ref[...], 0.0)
    o_ref[...] = jnp.dot(h, w_ref[...], preferred_element_type=jnp.float32)


def _mm(a, w):
    M, K = a.shape
    N = w.shape[1]
    G = 10
    return pl.pallas_call(
        lambda a_ref, w_ref, o_ref: o_ref.__setitem__(
            (...,), jnp.dot(a_ref[...], w_ref[...],
                            preferred_element_type=jnp.float32)),
        out_shape=jax.ShapeDtypeStruct((M, N), jnp.float32),
        grid=(G,),
        in_specs=[pl.BlockSpec((M // G, K), lambda i: (i, 0)),
                  pl.BlockSpec((K, N), lambda i: (0, 0))],
        out_specs=pl.BlockSpec((M // G, N), lambda i: (i, 0)),
    )(a, w)


def _relu_mm(s, b, w):
    M, K = s.shape
    N = w.shape[1]
    G = 10
    return pl.pallas_call(
        _relu_mm_kernel,
        out_shape=jax.ShapeDtypeStruct((M, N), jnp.float32),
        grid=(G,),
        in_specs=[pl.BlockSpec((M // G, K), lambda i: (i, 0)),
                  pl.BlockSpec((K,), lambda i: (0,)),
                  pl.BlockSpec((K, N), lambda i: (0, 0))],
        out_specs=pl.BlockSpec((M // G, N), lambda i: (i, 0)),
    )(s, b, w)


def _mlp_kernel(s_ref, b3_ref, m1_ref, mb1_ref, m2_ref, mb2_ref, o_ref):
    h3 = jnp.maximum(s_ref[...] + b3_ref[...], 0.0)
    z = jnp.dot(h3, m1_ref[...], preferred_element_type=jnp.float32) + mb1_ref[...]
    z = jnp.maximum(z, 0.0)
    o_ref[...] = jnp.dot(z, m2_ref[...], preferred_element_type=jnp.float32) + mb2_ref[...]


def _mlp(s3, b3, M1, mb1, M2, mb2):
    M = s3.shape[0]
    G = 10
    return pl.pallas_call(
        _mlp_kernel,
        out_shape=jax.ShapeDtypeStruct((M, 512), jnp.float32),
        grid=(G,),
        in_specs=[pl.BlockSpec((M // G, 64), lambda i: (i, 0)),
                  pl.BlockSpec((64,), lambda i: (0,)),
                  pl.BlockSpec((64, 512), lambda i: (0, 0)),
                  pl.BlockSpec((512,), lambda i: (0,)),
                  pl.BlockSpec((512, 512), lambda i: (0, 0)),
                  pl.BlockSpec((512,), lambda i: (0,))],
        out_specs=pl.BlockSpec((M // G, 512), lambda i: (i, 0)),
    )(s3, b3, M1, mb1, M2, mb2)


def _score_kernel(agg_ref, h_ref, wrel_ref, brel_ref, wroot_ref, o_ref):
    u = jnp.dot(agg_ref[...], wrel_ref[...],
                preferred_element_type=jnp.float32) + brel_ref[...]
    u = u + jnp.dot(h_ref[...], wroot_ref[...],
                    preferred_element_type=jnp.float32)
    o_ref[...] = jnp.tanh(u)


def _score(agg, h, Wrel, brel, Wroot):
    M = agg.shape[0]
    G = 10
    return pl.pallas_call(
        _score_kernel,
        out_shape=jax.ShapeDtypeStruct((M, 1), jnp.float32),
        grid=(G,),
        in_specs=[pl.BlockSpec((M // G, 512), lambda i: (i, 0)),
                  pl.BlockSpec((M // G, 512), lambda i: (i, 0)),
                  pl.BlockSpec((512, 1), lambda i: (0, 0)),
                  pl.BlockSpec((1,), lambda i: (0,)),
                  pl.BlockSpec((512, 1), lambda i: (0, 0))],
        out_specs=pl.BlockSpec((M // G, 1), lambda i: (i, 0)),
    )(agg, h, Wrel, brel, Wroot)


def _topk_kernel(score_ref, batch_ref, w_ref):
    score = score_ref[...]
    batch = batch_ref[...]
    gids = lax.broadcasted_iota(jnp.int32, (1, B), 1)
    O = batch == gids

    counts = jnp.sum(O.astype(jnp.float32), axis=0, keepdims=True)
    kf = jnp.ceil(0.5 * counts)

    ibits = lax.bitcast_convert_type(score, jnp.int32)
    key = jnp.where(ibits >= 0, ibits, ibits ^ jnp.int32(0x7FFFFFFF))

    def count_gt(t):
        ind = jnp.logical_and(O, key > t)
        return jnp.sum(ind.astype(jnp.float32), axis=0, keepdims=True)

    def body(it, lohi):
        lo, hi = lohi
        mid = (lo >> 1) + (hi >> 1) + (lo & hi & 1)
        pred = count_gt(mid) < kf
        return (jnp.where(pred, lo, mid + 1), jnp.where(pred, mid, hi))

    lo = jnp.full((1, B), -2**31, jnp.int32)
    hi = jnp.full((1, B), 2**31 - 1, jnp.int32)
    lo, hi = lax.fori_loop(0, 32, body, (lo, hi))
    v = lo

    gtM = jnp.logical_and(O, key > v)
    tieM = jnp.logical_and(O, key == v)
    gcnt = jnp.sum(gtM.astype(jnp.float32), axis=0, keepdims=True)
    r = kf - gcnt

    pos = lax.broadcasted_iota(jnp.int32, (NPAD, 1), 0)

    def count_tie_lt(t):
        ind = jnp.logical_and(tieM, pos < t)
        return jnp.sum(ind.astype(jnp.float32), axis=0, keepdims=True)

    def body2(it, lohi):
        lo, hi = lohi
        mid = (lo + hi) >> 1
        pred = count_tie_lt(mid) >= r
        return (jnp.where(pred, lo, mid + 1), jnp.where(pred, mid, hi))

    lo2 = jnp.zeros((1, B), jnp.int32)
    hi2 = jnp.full((1, B), 16384, jnp.int32)
    lo2, hi2 = lax.fori_loop(0, 15, body2, (lo2, hi2))
    theta = lo2

    selM = jnp.logical_or(gtM, jnp.logical_and(tieM, pos < theta))
    sel = jnp.any(selM, axis=1, keepdims=True)
    w_ref[...] = score * sel.astype(jnp.float32)


def _topk(score_pad, batch_pad):
    return pl.pallas_call(
        _topk_kernel,
        out_shape=jax.ShapeDtypeStruct((NPAD, 1), jnp.float32),
    )(score_pad, batch_pad)


def _pool_kernel(h_ref, w_ref, batch_ref, p1_ref, pb1_ref, p2_ref, pb2_ref,
                 o_ref, acc_ref):
    i = pl.program_id(0)

    @pl.when(i == 0)
    def _():
        acc_ref[...] = jnp.zeros_like(acc_ref)

    gids = lax.broadcasted_iota(jnp.int32, (1, B), 1)
    O = (batch_ref[...] == gids).astype(jnp.float32)
    hw = h_ref[...] * w_ref[...]
    acc_ref[...] += lax.dot_general(
        O, hw, (((0,), (0,)), ((), ())),
        precision=lax.Precision.HIGHEST,
        preferred_element_type=jnp.float32)

    @pl.when(i == pl.num_programs(0) - 1)
    def _():
        pooled = acc_ref[...]
        o1 = jnp.dot(pooled, p1_ref[...],
                     preferred_element_type=jnp.float32) + pb1_ref[...]
        o1 = jnp.maximum(o1, 0.0)
        o_ref[...] = jnp.dot(o1, p2_ref[...],
                             preferred_element_type=jnp.float32) + pb2_ref[...]


def _pool_head(h_pad, w_pad, batch_pad, P1, pb1, P2, pb2):
    G = 10
    blk = NPAD // G
    return pl.pallas_call(
        _pool_kernel,
        out_shape=jax.ShapeDtypeStruct((B, 256), jnp.float32),
        grid=(G,),
        in_specs=[pl.BlockSpec((blk, 512), lambda i: (i, 0)),
                  pl.BlockSpec((blk, 1), lambda i: (i, 0)),
                  pl.BlockSpec((blk, 1), lambda i: (i, 0)),
                  pl.BlockSpec((512, 512), lambda i: (0, 0)),
                  pl.BlockSpec((512,), lambda i: (0,)),
                  pl.BlockSpec((512, 256), lambda i: (0, 0)),
                  pl.BlockSpec((256,), lambda i: (0,))],
        out_specs=pl.BlockSpec((B, 256), lambda i: (0, 0)),
        scratch_shapes=[pltpu_vmem((B, 512), jnp.float32)],
    )(h_pad, w_pad, batch_pad, P1, pb1, P2, pb2)


from jax.experimental.pallas import tpu as pltpu


def pltpu_vmem(shape, dtype):
    return pltpu.VMEM(shape, dtype)


def kernel(x, edge_index, batch, W1, b1, W2, b2, W3, b3, M1, mb1, M2, mb2,
           Wrel, brel, Wroot, P1, pb1, P2, pb2):
    n = x.shape[0]
    src, dst = edge_index[0], edge_index[1]
    loops = jnp.arange(n)
    src2 = jnp.concatenate([src, loops])
    dst2 = jnp.concatenate([dst, loops])

    deg = jax.ops.segment_sum(jnp.ones(src2.shape[0], x.dtype), dst2,
                              num_segments=n)
    dinv = jnp.where(deg > 0, 1.0 / jnp.sqrt(deg), 0.0)
    norm = dinv[src2] * dinv[dst2]

    def conv_agg(Y):
        msg = Y[src2] * norm[:, None]
        return jax.ops.segment_sum(msg, dst2, num_segments=n)

    Y1 = _mm(x, W1)
    S1 = conv_agg(Y1)
    Y2 = _relu_mm(S1, b1, W2)
    S2 = conv_agg(Y2)
    Y3 = _relu_mm(S2, b2, W3)
    S3 = conv_agg(Y3)

    h = _mlp(S3, b3, M1, mb1, M2, mb2)

    agg = jax.ops.segment_sum(h[src], dst, num_segments=n)
    score = _score(agg, h, Wrel, brel, Wroot)

    pad = NPAD - n
    score_pad = jnp.concatenate([score, jnp.zeros((pad, 1), jnp.float32)])
    batch_pad = jnp.concatenate(
        [batch.astype(jnp.int32), jnp.full((pad,), 127, jnp.int32)]
    ).reshape(NPAD, 1)

    w_pad = _topk(score_pad, batch_pad)

    h_pad = jnp.concatenate([h, jnp.zeros((pad, 512), jnp.float32)])
    out = _pool_head(h_pad, w_pad, batch_pad, P1, pb1, P2, pb2)
    return out


# speedup vs baseline: 1.1597x; 1.0001x over previous
import functools

import jax
import jax.numpy as jnp
from jax import lax
from jax.experimental import pallas as pl
from jax.experimental.pallas import tpu as pltpu

N_NODES = 10000
NPAD = 10240
B = 64


def _mm_bias_kernel(a_ref, w_ref, b_ref, o_ref):
    o_ref[...] = jnp.dot(a_ref[...], w_ref[...],
                         preferred_element_type=jnp.float32) + b_ref[...]


def _relu_mm_kernel(s_ref, b_ref, w_ref, o_ref):
    h = jnp.maximum(s_ref[...] + b_ref[...], 0.0)
    o_ref[...] = jnp.dot(h, w_ref[...], preferred_element_type=jnp.float32)


def _mm(a, w):
    M, K = a.shape
    N = w.shape[1]
    G = 10
    return pl.pallas_call(
        lambda a_ref, w_ref, o_ref: o_ref.__setitem__(
            (...,), jnp.dot(a_ref[...], w_ref[...],
                            preferred_element_type=jnp.float32)),
        out_shape=jax.ShapeDtypeStruct((M, N), jnp.float32),
        grid=(G,),
        in_specs=[pl.BlockSpec((M // G, K), lambda i: (i, 0)),
                  pl.BlockSpec((K, N), lambda i: (0, 0))],
        out_specs=pl.BlockSpec((M // G, N), lambda i: (i, 0)),
    )(a, w)


def _relu_mm(s, b, w):
    M, K = s.shape
    N = w.shape[1]
    G = 10
    return pl.pallas_call(
        _relu_mm_kernel,
        out_shape=jax.ShapeDtypeStruct((M, N), jnp.float32),
        grid=(G,),
        in_specs=[pl.BlockSpec((M // G, K), lambda i: (i, 0)),
                  pl.BlockSpec((K,), lambda i: (0,)),
                  pl.BlockSpec((K, N), lambda i: (0, 0))],
        out_specs=pl.BlockSpec((M // G, N), lambda i: (i, 0)),
    )(s, b, w)


def _mlp_kernel(s_ref, b3_ref, m1_ref, mb1_ref, m2_ref, mb2_ref, o_ref):
    h3 = jnp.maximum(s_ref[...] + b3_ref[...], 0.0)
    z = jnp.dot(h3, m1_ref[...], preferred_element_type=jnp.float32) + mb1_ref[...]
    z = jnp.maximum(z, 0.0)
    o_ref[...] = jnp.dot(z, m2_ref[...], preferred_element_type=jnp.float32) + mb2_ref[...]


def _mlp(s3, b3, M1, mb1, M2, mb2):
    M = s3.shape[0]
    G = 10
    return pl.pallas_call(
        _mlp_kernel,
        out_shape=jax.ShapeDtypeStruct((M, 512), jnp.float32),
        grid=(G,),
        in_specs=[pl.BlockSpec((M // G, 64), lambda i: (i, 0)),
                  pl.BlockSpec((64,), lambda i: (0,)),
                  pl.BlockSpec((64, 512), lambda i: (0, 0)),
                  pl.BlockSpec((512,), lambda i: (0,)),
                  pl.BlockSpec((512, 512), lambda i: (0, 0)),
                  pl.BlockSpec((512,), lambda i: (0,))],
        out_specs=pl.BlockSpec((M // G, 512), lambda i: (i, 0)),
    )(s3, b3, M1, mb1, M2, mb2)


def _score_kernel(agg_ref, h_ref, wrel_ref, brel_ref, wroot_ref, o_ref):
    u = jnp.dot(agg_ref[...], wrel_ref[...],
                preferred_element_type=jnp.float32) + brel_ref[...]
    u = u + jnp.dot(h_ref[...], wroot_ref[...],
                    preferred_element_type=jnp.float32)
    o_ref[...] = jnp.tanh(u)


def _score(agg, h, Wrel, brel, Wroot):
    M = agg.shape[0]
    G = 10
    return pl.pallas_call(
        _score_kernel,
        out_shape=jax.ShapeDtypeStruct((M, 1), jnp.float32),
        grid=(G,),
        in_specs=[pl.BlockSpec((M // G, 512), lambda i: (i, 0)),
                  pl.BlockSpec((M // G, 512), lambda i: (i, 0)),
                  pl.BlockSpec((512, 1), lambda i: (0, 0)),
                  pl.BlockSpec((1,), lambda i: (0,)),
                  pl.BlockSpec((512, 1), lambda i: (0, 0))],
        out_specs=pl.BlockSpec((M // G, 1), lambda i: (i, 0)),
    )(agg, h, Wrel, brel, Wroot)


def _topk_kernel(score_ref, batch_ref, w_ref):
    score = score_ref[...]
    batch = batch_ref[...]
    gids = lax.broadcasted_iota(jnp.int32, (1, B), 1)
    O = batch == gids

    counts = jnp.sum(O.astype(jnp.float32), axis=0, keepdims=True)
    kf = jnp.ceil(0.5 * counts)

    ibits = lax.bitcast_convert_type(score, jnp.int32)
    key = jnp.where(ibits >= 0, ibits, ibits ^ jnp.int32(0x7FFFFFFF))

    def count_gt(t):
        ind = jnp.logical_and(O, key > t)
        return jnp.sum(ind.astype(jnp.float32), axis=0, keepdims=True)

    def body(it, lohi):
        lo, hi = lohi
        mid = (lo >> 1) + (hi >> 1) + (lo & hi & 1)
        pred = count_gt(mid) < kf
        return (jnp.where(pred, lo, mid + 1), jnp.where(pred, mid, hi))

    lo = jnp.full((1, B), -2**31, jnp.int32)
    hi = jnp.full((1, B), 2**31 - 1, jnp.int32)
    lo, hi = lax.fori_loop(0, 32, body, (lo, hi))
    v = lo

    gtM = jnp.logical_and(O, key > v)
    tieM = jnp.logical_and(O, key == v)
    gcnt = jnp.sum(gtM.astype(jnp.float32), axis=0, keepdims=True)
    r = kf - gcnt

    pos = lax.broadcasted_iota(jnp.int32, (NPAD, 1), 0)

    def count_tie_lt(t):
        ind = jnp.logical_and(tieM, pos < t)
        return jnp.sum(ind.astype(jnp.float32), axis=0, keepdims=True)

    def body2(it, lohi):
        lo, hi = lohi
        mid = (lo + hi) >> 1
        pred = count_tie_lt(mid) >= r
        return (jnp.where(pred, lo, mid + 1), jnp.where(pred, mid, hi))

    lo2 = jnp.zeros((1, B), jnp.int32)
    hi2 = jnp.full((1, B), 16384, jnp.int32)
    lo2, hi2 = lax.fori_loop(0, 15, body2, (lo2, hi2))
    theta = lo2

    selM = jnp.logical_or(gtM, jnp.logical_and(tieM, pos < theta))
    sel = jnp.any(selM, axis=1, keepdims=True)
    w_ref[...] = score * sel.astype(jnp.float32)


def _topk(score_pad, batch_pad):
    return pl.pallas_call(
        _topk_kernel,
        out_shape=jax.ShapeDtypeStruct((NPAD, 1), jnp.float32),
    )(score_pad, batch_pad)


def _pool_kernel(h_ref, w_ref, batch_ref, p1_ref, pb1_ref, p2_ref, pb2_ref,
                 o_ref, acc_ref):
    i = pl.program_id(0)

    @pl.when(i == 0)
    def _():
        acc_ref[...] = jnp.zeros_like(acc_ref)

    gids = lax.broadcasted_iota(jnp.int32, (1, B), 1)
    O = (batch_ref[...] == gids).astype(jnp.float32)
    hw = h_ref[...] * w_ref[...]
    acc_ref[...] += lax.dot_general(
        O, hw, (((0,), (0,)), ((), ())),
        precision=lax.Precision.HIGHEST,
        preferred_element_type=jnp.float32)

    @pl.when(i == pl.num_programs(0) - 1)
    def _():
        pooled = acc_ref[...]
        o1 = jnp.dot(pooled, p1_ref[...],
                     preferred_element_type=jnp.float32) + pb1_ref[...]
        o1 = jnp.maximum(o1, 0.0)
        o_ref[...] = jnp.dot(o1, p2_ref[...],
                             preferred_element_type=jnp.float32) + pb2_ref[...]


def _pool_head(h_pad, w_pad, batch_pad, P1, pb1, P2, pb2):
    G = 10
    blk = NPAD // G
    return pl.pallas_call(
        _pool_kernel,
        out_shape=jax.ShapeDtypeStruct((B, 256), jnp.float32),
        grid=(G,),
        in_specs=[pl.BlockSpec((blk, 512), lambda i: (i, 0)),
                  pl.BlockSpec((blk, 1), lambda i: (i, 0)),
                  pl.BlockSpec((blk, 1), lambda i: (i, 0)),
                  pl.BlockSpec((512, 512), lambda i: (0, 0)),
                  pl.BlockSpec((512,), lambda i: (0,)),
                  pl.BlockSpec((512, 256), lambda i: (0, 0)),
                  pl.BlockSpec((256,), lambda i: (0,))],
        out_specs=pl.BlockSpec((B, 256), lambda i: (0, 0)),
        scratch_shapes=[pltpu.VMEM((B, 512), jnp.float32)],
    )(h_pad, w_pad, batch_pad, P1, pb1, P2, pb2)


def kernel(x, edge_index, batch, W1, b1, W2, b2, W3, b3, M1, mb1, M2, mb2,
           Wrel, brel, Wroot, P1, pb1, P2, pb2):
    n = x.shape[0]
    src, dst = edge_index[0], edge_index[1]
    loops = jnp.arange(n)
    src2 = jnp.concatenate([src, loops])
    dst2 = jnp.concatenate([dst, loops])

    deg = jax.ops.segment_sum(jnp.ones(src2.shape[0], x.dtype), dst2,
                              num_segments=n)
    dinv = jnp.where(deg > 0, 1.0 / jnp.sqrt(deg), 0.0)
    norm = dinv[src2] * dinv[dst2]

    def conv_agg(Y):
        msg = Y[src2] * norm[:, None]
        return jax.ops.segment_sum(msg, dst2, num_segments=n)

    Y1 = _mm(x, W1)
    S1 = conv_agg(Y1)
    Y2 = _relu_mm(S1, b1, W2)
    S2 = conv_agg(Y2)
    Y3 = _relu_mm(S2, b2, W3)
    S3 = conv_agg(Y3)

    h = _mlp(S3, b3, M1, mb1, M2, mb2)

    agg = jax.ops.segment_sum(h[src], dst, num_segments=n)
    score = _score(agg, h, Wrel, brel, Wroot)

    pad = NPAD - n
    score_pad = jnp.concatenate([score, jnp.zeros((pad, 1), jnp.float32)])
    batch_pad = jnp.concatenate(
        [batch.astype(jnp.int32), jnp.full((pad,), 127, jnp.int32)]
    ).reshape(NPAD, 1)

    w_pad = _topk(score_pad, batch_pad)

    h_pad = jnp.concatenate([h, jnp.zeros((pad, 512), jnp.float32)])
    out = _pool_head(h_pad, w_pad, batch_pad, P1, pb1, P2, pb2)
    return out
